# Initial kernel scaffold; baseline (speedup 1.0000x reference)
import jax
import jax.numpy as jnp
from jax.experimental import pallas as pl


def kernel(input_tensor, emb_weight):
    raise NotImplementedError("write your pallas kernel here")


# speedup vs baseline: 1.1004x; 1.1004x over previous
import functools

import jax
import jax.numpy as jnp
from jax import lax
from jax.experimental import pallas as pl
from jax.experimental.pallas import tpu as pltpu
from jax.experimental.pallas import tpu_sc as plsc

NUM_EMB = 1000000
DIM = 32
HASH_MULT_I32 = -1640531535

ROWS = 16384
COLS = 100
TOTAL = ROWS * COLS

NUM_CORES = 2
NUM_SUBCORES = 16
NW = NUM_CORES * NUM_SUBCORES
PER_W = TOTAL // NW
CHUNK = 1024
NCHUNK = PER_W // CHUNK


def _hash_body(x_ref, o_ref):
    h = x_ref[...] * jnp.int32(HASH_MULT_I32)
    hi = lax.shift_right_logical(h, 16)
    lo = jnp.bitwise_and(h, jnp.int32(0xFFFF))
    t = (hi * jnp.int32(256)) % jnp.int32(NUM_EMB)
    o_ref[...] = (t * jnp.int32(256) + lo) % jnp.int32(NUM_EMB)


def _hash_tc(x2d):
    return pl.pallas_call(
        _hash_body,
        out_shape=jax.ShapeDtypeStruct(x2d.shape, jnp.int32),
    )(x2d)


def _sc_gather_body(idx_hbm, table_hbm, out_hbm, idx_v, rows_v, sem):
    wid = lax.axis_index("s") * NUM_CORES + lax.axis_index("c")
    base = wid * PER_W

    def chunk_body(g, carry):
        off = base + g * CHUNK
        pltpu.sync_copy(idx_hbm.at[pl.ds(off, CHUNK)], idx_v)
        pltpu.async_copy(table_hbm.at[idx_v], rows_v, sem).wait()
        pltpu.sync_copy(rows_v, out_hbm.at[pl.ds(off, CHUNK)])
        return carry

    lax.fori_loop(0, NCHUNK, chunk_body, 0)


_sc_gather = functools.partial(
    pl.kernel,
    out_type=jax.ShapeDtypeStruct((TOTAL, DIM), jnp.float32),
    mesh=plsc.VectorSubcoreMesh(core_axis_name="c", subcore_axis_name="s"),
    scratch_types=[
        pltpu.VMEM((CHUNK,), jnp.int32),
        pltpu.VMEM((CHUNK, DIM), jnp.float32),
        pltpu.SemaphoreType.DMA,
    ],
    compiler_params=pltpu.CompilerParams(use_tc_tiling_on_sc=False),
)(_sc_gather_body)


def kernel(input_tensor, emb_weight):
    idx = _hash_tc(input_tensor.reshape(TOTAL // 128, 128)).reshape(TOTAL)
    out = _sc_gather(idx, emb_weight)
    return out.reshape(ROWS, COLS, DIM)


# speedup vs baseline: 1.1103x; 1.0089x over previous
import functools

import jax
import jax.numpy as jnp
from jax import lax
from jax.experimental import pallas as pl
from jax.experimental.pallas import tpu as pltpu
from jax.experimental.pallas import tpu_sc as plsc

NUM_EMB = 1000000
DIM = 32
HASH_MULT_I32 = -1640531535

ROWS = 16384
COLS = 100
TOTAL = ROWS * COLS

NUM_CORES = 2
NUM_SUBCORES = 16
NW = NUM_CORES * NUM_SUBCORES
PER_W = TOTAL // NW
CHUNK = 1024
NCHUNK = PER_W // CHUNK


def _hash_body(x_ref, o_ref):
    h = x_ref[...] * jnp.int32(HASH_MULT_I32)
    hi = lax.shift_right_logical(h, 16)
    lo = jnp.bitwise_and(h, jnp.int32(0xFFFF))
    t = (hi * jnp.int32(256)) % jnp.int32(NUM_EMB)
    o_ref[...] = (t * jnp.int32(256) + lo) % jnp.int32(NUM_EMB)


def _hash_tc(x2d):
    return pl.pallas_call(
        _hash_body,
        out_shape=jax.ShapeDtypeStruct(x2d.shape, jnp.int32),
    )(x2d)


NBUF = 2


def _sc_gather_body(idx_hbm, table_hbm, out_hbm, idx_v, rows_v,
                    idx_sem, gat_sem, out_sem):
    wid = lax.axis_index("s") * NUM_CORES + lax.axis_index("c")
    base = wid * PER_W

    def start_idx(g, b):
        pltpu.async_copy(idx_hbm.at[pl.ds(base + g * CHUNK, CHUNK)],
                         idx_v.at[b], idx_sem.at[b])

    def wait_idx(b):
        pltpu.make_async_copy(idx_hbm.at[pl.ds(base, CHUNK)],
                              idx_v.at[b], idx_sem.at[b]).wait()

    def start_gather(b):
        pltpu.async_copy(table_hbm.at[idx_v.at[b]], rows_v.at[b],
                         gat_sem.at[b])

    def wait_gather(b):
        pltpu.make_async_copy(table_hbm.at[idx_v.at[b]], rows_v.at[b],
                              gat_sem.at[b]).wait()

    def start_out(g, b):
        pltpu.async_copy(rows_v.at[b],
                         out_hbm.at[pl.ds(base + g * CHUNK, CHUNK)],
                         out_sem.at[b])

    def wait_out(b):
        pltpu.make_async_copy(rows_v.at[b],
                              out_hbm.at[pl.ds(base, CHUNK)],
                              out_sem.at[b]).wait()

    for b in range(NBUF):
        start_idx(b, b)

    def outer(j, carry):
        for b in range(NBUF):
            g = j * NBUF + b

            @pl.when(j > 0)
            def _():
                wait_out(b)

            wait_idx(b)
            start_gather(b)

            bp = (b - 1) % NBUF

            def retire(g_prev, bp=bp):
                wait_gather(bp)
                start_out(g_prev, bp)

                @pl.when(g_prev + NBUF < NCHUNK)
                def _():
                    start_idx(g_prev + NBUF, bp)

            if b == 0:
                @pl.when(j > 0)
                def _():
                    retire(j * NBUF - 1)
            else:
                retire(g - 1)
        return carry

    lax.fori_loop(0, NCHUNK // NBUF, outer, 0)

    b_last = (NCHUNK - 1) % NBUF
    wait_gather(b_last)
    start_out(NCHUNK - 1, b_last)
    for b in range(NBUF):
        wait_out(b)


_sc_gather = functools.partial(
    pl.kernel,
    out_type=jax.ShapeDtypeStruct((TOTAL, DIM), jnp.float32),
    mesh=plsc.VectorSubcoreMesh(core_axis_name="c", subcore_axis_name="s"),
    scratch_types=[
        pltpu.VMEM((NBUF, CHUNK), jnp.int32),
        pltpu.VMEM((NBUF, CHUNK, DIM), jnp.float32),
        pltpu.SemaphoreType.DMA((NBUF,)),
        pltpu.SemaphoreType.DMA((NBUF,)),
        pltpu.SemaphoreType.DMA((NBUF,)),
    ],
    compiler_params=pltpu.CompilerParams(use_tc_tiling_on_sc=False),
)(_sc_gather_body)


def kernel(input_tensor, emb_weight):
    idx = _hash_tc(input_tensor.reshape(TOTAL // 128, 128)).reshape(TOTAL)
    out = _sc_gather(idx, emb_weight)
    return out.reshape(ROWS, COLS, DIM)


# speedup vs baseline: 3.3622x; 3.0283x over previous
import functools

import jax
import jax.numpy as jnp
from jax import lax
from jax.experimental import pallas as pl
from jax.experimental.pallas import tpu as pltpu
from jax.experimental.pallas import tpu_sc as plsc

NUM_EMB = 1000000
DIM = 32
HASH_MULT_I32 = -1640531535

ROWS = 16384
COLS = 100

NUM_CORES = 2
NUM_SUBCORES = 16
NW = NUM_CORES * NUM_SUBCORES
C = ROWS // NW
NCHUNK = COLS
NBUF = 2
L = 16


def _hash_body(x_ref, o_ref):
    h = x_ref[...] * jnp.int32(HASH_MULT_I32)
    hi = lax.shift_right_logical(h, 16)
    lo = jnp.bitwise_and(h, jnp.int32(0xFFFF))
    t = (hi * jnp.int32(256)) % jnp.int32(NUM_EMB)
    o_ref[...] = (t * jnp.int32(256) + lo) % jnp.int32(NUM_EMB)


def _hash_tc(x2d):
    return pl.pallas_call(
        _hash_body,
        out_shape=jax.ShapeDtypeStruct(x2d.shape, jnp.int32),
    )(x2d)


def _sc_gather_body(idx_hbm, table_hbm, out_hbm, idx_v, rows_v, trows_v,
                    idx_sem, gat_sem, out_sem):
    wid = lax.axis_index("s") * NUM_CORES + lax.axis_index("c")
    sbase = wid * C

    def start_idx(j, b):
        pltpu.async_copy(idx_hbm.at[j, pl.ds(sbase, C)],
                         idx_v.at[b], idx_sem.at[b])

    def wait_idx(b):
        pltpu.make_async_copy(idx_hbm.at[0, pl.ds(sbase, C)],
                              idx_v.at[b], idx_sem.at[b]).wait()

    def start_gather(b):
        pltpu.async_copy(table_hbm.at[idx_v.at[b]], rows_v.at[b],
                         gat_sem.at[b])

    def wait_gather(b):
        pltpu.make_async_copy(table_hbm.at[idx_v.at[b]], rows_v.at[b],
                              gat_sem.at[b]).wait()

    def start_out(j, b):
        pltpu.async_copy(trows_v.at[b],
                         out_hbm.at[j, :, pl.ds(sbase, C)],
                         out_sem.at[b])

    def wait_out(b):
        pltpu.make_async_copy(trows_v.at[b],
                              out_hbm.at[0, :, pl.ds(sbase, C)],
                              out_sem.at[b]).wait()

    def transpose(b):
        rows = rows_v.at[b]
        trows = trows_v.at[b]
        iota = lax.iota(jnp.int32, L)

        def blk(i0g, carry):
            rvec = iota + i0g * L
            for d in range(DIM):
                cvec = jnp.full((L,), d, jnp.int32)
                vals = plsc.load_gather(rows, [rvec, cvec])
                trows[d, pl.ds(i0g * L, L)] = vals
            return carry

        lax.fori_loop(0, C // L, blk, 0)

    for b in range(NBUF):
        start_idx(b, b)

    def outer(jj, carry):
        for b in range(NBUF):
            g = jj * NBUF + b

            @pl.when(jj > 0)
            def _():
                wait_out(b)

            wait_idx(b)
            start_gather(b)

            bp = (b - 1) % NBUF

            def retire(g_prev, bp=bp):
                wait_gather(bp)

                @pl.when(g_prev + NBUF < NCHUNK)
                def _():
                    start_idx(g_prev + NBUF, bp)

                transpose(bp)
                start_out(g_prev, bp)

            if b == 0:
                @pl.when(jj > 0)
                def _():
                    retire(jj * NBUF - 1)
            else:
                retire(g - 1)
        return carry

    lax.fori_loop(0, NCHUNK // NBUF, outer, 0)

    b_last = (NCHUNK - 1) % NBUF
    wait_gather(b_last)
    transpose(b_last)
    start_out(NCHUNK - 1, b_last)
    for b in range(NBUF):
        wait_out(b)


_sc_gather = functools.partial(
    pl.kernel,
    out_type=jax.ShapeDtypeStruct((COLS, DIM, ROWS), jnp.float32),
    mesh=plsc.VectorSubcoreMesh(core_axis_name="c", subcore_axis_name="s"),
    scratch_types=[
        pltpu.VMEM((NBUF, C), jnp.int32),
        pltpu.VMEM((NBUF, C, DIM), jnp.float32),
        pltpu.VMEM((NBUF, DIM, C), jnp.float32),
        pltpu.SemaphoreType.DMA((NBUF,)),
        pltpu.SemaphoreType.DMA((NBUF,)),
        pltpu.SemaphoreType.DMA((NBUF,)),
    ],
    compiler_params=pltpu.CompilerParams(use_tc_tiling_on_sc=False,
                                         needs_layout_passes=False),
)(_sc_gather_body)


def kernel(input_tensor, emb_weight):
    idx_t = _hash_tc(input_tensor.T)
    out_t = _sc_gather(idx_t, emb_weight)
    return jnp.transpose(out_t, (2, 0, 1))


# speedup vs baseline: 6.7323x; 2.0024x over previous
import functools

import jax
import jax.numpy as jnp
from jax import lax
from jax.experimental import pallas as pl
from jax.experimental.pallas import tpu as pltpu
from jax.experimental.pallas import tpu_sc as plsc

NUM_EMB = 1000000
DIM = 32
HASH_MULT_I32 = -1640531535

ROWS = 16384
COLS = 100

NUM_CORES = 2
NUM_SUBCORES = 16
NW = NUM_CORES * NUM_SUBCORES
C = ROWS // NW
NCHUNK = COLS
NBUF = 2
L = 16


def _hash_body(x_ref, o_ref):
    h = x_ref[...] * jnp.int32(HASH_MULT_I32)
    hi = lax.shift_right_logical(h, 16)
    lo = jnp.bitwise_and(h, jnp.int32(0xFFFF))
    t = (hi * jnp.int32(256)) % jnp.int32(NUM_EMB)
    o_ref[...] = (t * jnp.int32(256) + lo) % jnp.int32(NUM_EMB)


def _hash_tc(x2d):
    return pl.pallas_call(
        _hash_body,
        out_shape=jax.ShapeDtypeStruct(x2d.shape, jnp.int32),
    )(x2d)


def _sc_gather_body(idx_hbm, table_hbm, out_hbm, idx_v, rows_v, trows_v,
                    idx_sem, gat_sem, out_sem):
    wid = lax.axis_index("s") * NUM_CORES + lax.axis_index("c")
    sbase = wid * C

    def start_idx(j, b):
        pltpu.async_copy(idx_hbm.at[j, pl.ds(sbase, C)],
                         idx_v.at[b], idx_sem.at[b])

    def wait_idx(b):
        pltpu.make_async_copy(idx_hbm.at[0, pl.ds(sbase, C)],
                              idx_v.at[b], idx_sem.at[b]).wait()

    def start_gather(b):
        pltpu.async_copy(table_hbm.at[idx_v.at[b]], rows_v.at[b],
                         gat_sem.at[b])

    def wait_gather(b):
        pltpu.make_async_copy(table_hbm.at[idx_v.at[b]], rows_v.at[b],
                              gat_sem.at[b]).wait()

    def start_out(j, b):
        pltpu.async_copy(trows_v.at[b, :, pl.ds(0, C)],
                         out_hbm.at[j, :, pl.ds(sbase, C)],
                         out_sem.at[b])

    def wait_out(b):
        pltpu.make_async_copy(trows_v.at[b, :, pl.ds(0, C)],
                              out_hbm.at[0, :, pl.ds(sbase, C)],
                              out_sem.at[b]).wait()

    def transpose(b):
        rows = rows_v.at[b]
        trows = trows_v.at[b]
        d_lo = lax.iota(jnp.int32, L)
        d_hi = d_lo + L

        def blk(i0, carry):
            for u in range(4):
                i = i0 * 4 + u
                col = jnp.full((L,), 0, jnp.int32) + i
                v0 = rows[i, pl.ds(0, L)]
                v1 = rows[i, pl.ds(L, L)]
                plsc.store_scatter(trows, [d_lo, col], v0)
                plsc.store_scatter(trows, [d_hi, col], v1)
            return carry

        lax.fori_loop(0, C // 4, blk, 0)

    for b in range(NBUF):
        start_idx(b, b)

    def outer(jj, carry):
        for b in range(NBUF):
            g = jj * NBUF + b

            @pl.when(jj > 0)
            def _():
                wait_out(b)

            wait_idx(b)
            start_gather(b)

            bp = (b - 1) % NBUF

            def retire(g_prev, bp=bp):
                wait_gather(bp)

                @pl.when(g_prev + NBUF < NCHUNK)
                def _():
                    start_idx(g_prev + NBUF, bp)

                transpose(bp)
                start_out(g_prev, bp)

            if b == 0:
                @pl.when(jj > 0)
                def _():
                    retire(jj * NBUF - 1)
            else:
                retire(g - 1)
        return carry

    lax.fori_loop(0, NCHUNK // NBUF, outer, 0)

    b_last = (NCHUNK - 1) % NBUF
    wait_gather(b_last)
    transpose(b_last)
    start_out(NCHUNK - 1, b_last)
    for b in range(NBUF):
        wait_out(b)


_sc_gather = functools.partial(
    pl.kernel,
    out_type=jax.ShapeDtypeStruct((COLS, DIM, ROWS), jnp.float32),
    mesh=plsc.VectorSubcoreMesh(core_axis_name="c", subcore_axis_name="s"),
    scratch_types=[
        pltpu.VMEM((NBUF, C), jnp.int32),
        pltpu.VMEM((NBUF, C, DIM), jnp.float32),
        pltpu.VMEM((NBUF, DIM, C + 1), jnp.float32),
        pltpu.SemaphoreType.DMA((NBUF,)),
        pltpu.SemaphoreType.DMA((NBUF,)),
        pltpu.SemaphoreType.DMA((NBUF,)),
    ],
    compiler_params=pltpu.CompilerParams(use_tc_tiling_on_sc=False,
                                         needs_layout_passes=False),
)(_sc_gather_body)


def kernel(input_tensor, emb_weight):
    idx_t = _hash_tc(input_tensor.T)
    out_t = _sc_gather(idx_t, emb_weight)
    return jnp.transpose(out_t, (2, 0, 1))


# speedup vs baseline: 6.8007x; 1.0102x over previous
import functools

import jax
import jax.numpy as jnp
from jax import lax
from jax.experimental import pallas as pl
from jax.experimental.pallas import tpu as pltpu
from jax.experimental.pallas import tpu_sc as plsc

NUM_EMB = 1000000
DIM = 32
HASH_MULT_I32 = -1640531535

ROWS = 16384
COLS = 100

NUM_CORES = 2
NUM_SUBCORES = 16
NW = NUM_CORES * NUM_SUBCORES
C = ROWS // NW
NCHUNK = COLS
NBUF = 2
L = 16


def _hash_body(x_ref, o_ref):
    h = x_ref[...] * jnp.int32(HASH_MULT_I32)
    hi = lax.shift_right_logical(h, 16)
    lo = jnp.bitwise_and(h, jnp.int32(0xFFFF))
    t = (hi * jnp.int32(256)) % jnp.int32(NUM_EMB)
    r = (t * jnp.int32(256) + lo) % jnp.int32(NUM_EMB)
    o_ref[...] = r * jnp.int32(4)


def _hash_tc(x2d):
    return pl.pallas_call(
        _hash_body,
        out_shape=jax.ShapeDtypeStruct(x2d.shape, jnp.int32),
    )(x2d)


def _sc_gather_body(idx_hbm, table_hbm, out_hbm, idx_v, rows_v, trows_v,
                    idx_sem, gat_sem, out_sem):
    wid = lax.axis_index("s") * NUM_CORES + lax.axis_index("c")
    sbase = wid * C

    def start_idx(j, b):
        pltpu.async_copy(idx_hbm.at[j, pl.ds(sbase, C)],
                         idx_v.at[b], idx_sem.at[b])

    def wait_idx(b):
        pltpu.make_async_copy(idx_hbm.at[0, pl.ds(sbase, C)],
                              idx_v.at[b], idx_sem.at[b]).wait()

    def start_gather(b):
        pltpu.async_copy(table_hbm.at[idx_v.at[b]], rows_v.at[b],
                         gat_sem.at[b])

    def wait_gather(b):
        pltpu.make_async_copy(table_hbm.at[idx_v.at[b]], rows_v.at[b],
                              gat_sem.at[b]).wait()

    def start_out(j, b):
        pltpu.async_copy(trows_v.at[b, :, pl.ds(0, C)],
                         out_hbm.at[j, :, pl.ds(sbase, C)],
                         out_sem.at[b])

    def wait_out(b):
        pltpu.make_async_copy(trows_v.at[b, :, pl.ds(0, C)],
                              out_hbm.at[0, :, pl.ds(sbase, C)],
                              out_sem.at[b]).wait()

    def transpose(b):
        rows = rows_v.at[b]
        trows = trows_v.at[b]
        d_lo = lax.iota(jnp.int32, L)
        d_hi = d_lo + L

        def blk(i0, carry):
            for u in range(4):
                i = i0 * 4 + u
                col = jnp.full((L,), 0, jnp.int32) + i
                v0 = rows[i, pl.ds(0, L)]
                v1 = rows[i, pl.ds(L, L)]
                plsc.store_scatter(trows, [d_lo, col], v0)
                plsc.store_scatter(trows, [d_hi, col], v1)
            return carry

        lax.fori_loop(0, C // 4, blk, 0)

    for b in range(NBUF):
        start_idx(b, b)

    def outer(jj, carry):
        for b in range(NBUF):
            g = jj * NBUF + b

            @pl.when(jj > 0)
            def _():
                wait_out(b)

            wait_idx(b)
            start_gather(b)

            bp = (b - 1) % NBUF

            def retire(g_prev, bp=bp):
                wait_gather(bp)

                @pl.when(g_prev + NBUF < NCHUNK)
                def _():
                    start_idx(g_prev + NBUF, bp)

                transpose(bp)
                start_out(g_prev, bp)

            if b == 0:
                @pl.when(jj > 0)
                def _():
                    retire(jj * NBUF - 1)
            else:
                retire(g - 1)
        return carry

    lax.fori_loop(0, NCHUNK // NBUF, outer, 0)

    b_last = (NCHUNK - 1) % NBUF
    wait_gather(b_last)
    transpose(b_last)
    start_out(NCHUNK - 1, b_last)
    for b in range(NBUF):
        wait_out(b)


_sc_gather = functools.partial(
    pl.kernel,
    out_type=jax.ShapeDtypeStruct((COLS, DIM, ROWS), jnp.float32),
    mesh=plsc.VectorSubcoreMesh(core_axis_name="c", subcore_axis_name="s"),
    scratch_types=[
        pltpu.VMEM((NBUF, C), jnp.int32),
        pltpu.VMEM((NBUF, C, DIM), jnp.float32),
        pltpu.VMEM((NBUF, DIM, C + 1), jnp.float32),
        pltpu.SemaphoreType.DMA((NBUF,)),
        pltpu.SemaphoreType.DMA((NBUF,)),
        pltpu.SemaphoreType.DMA((NBUF,)),
    ],
    compiler_params=pltpu.CompilerParams(use_tc_tiling_on_sc=False,
                                         needs_layout_passes=False),
)(_sc_gather_body)


def kernel(input_tensor, emb_weight):
    idx_t = _hash_tc(input_tensor.T)
    table4 = jnp.pad(emb_weight, ((0, 0), (0, 96))).reshape(4 * NUM_EMB, DIM)
    out_t = _sc_gather(idx_t, table4)
    return jnp.transpose(out_t, (2, 0, 1))


# speedup vs baseline: 10.3802x; 1.5263x over previous
import functools

import jax
import jax.numpy as jnp
from jax import lax
from jax.experimental import pallas as pl
from jax.experimental.pallas import tpu as pltpu
from jax.experimental.pallas import tpu_sc as plsc

NUM_EMB = 1000000
DIM = 32
HASH_MULT_I32 = -1640531535

ROWS = 16384
COLS = 100

NUM_CORES = 2
NUM_SUBCORES = 16
NW = NUM_CORES * NUM_SUBCORES
C = ROWS // NW
NCHUNK = COLS
NBUF = 2
L = 16


def _hash_body(x_ref, o_ref):
    h = x_ref[...] * jnp.int32(HASH_MULT_I32)
    hi = lax.shift_right_logical(h, 16)
    lo = jnp.bitwise_and(h, jnp.int32(0xFFFF))
    t = (hi * jnp.int32(256)) % jnp.int32(NUM_EMB)
    r = (t * jnp.int32(256) + lo) % jnp.int32(NUM_EMB)
    o_ref[...] = r * jnp.int32(4)


def _hash_tc(x2d):
    return pl.pallas_call(
        _hash_body,
        out_shape=jax.ShapeDtypeStruct(x2d.shape, jnp.int32),
    )(x2d)


def _sc_gather_body(idx_hbm, table_hbm, out_hbm, idx_v, rows_v, trows_v,
                    idx_sem, gat_sem, out_sem):
    wid = lax.axis_index("s") * NUM_CORES + lax.axis_index("c")
    sbase = wid * C

    def start_idx(j, b):
        pltpu.async_copy(idx_hbm.at[j, pl.ds(sbase, C)],
                         idx_v.at[b], idx_sem.at[b])

    def wait_idx(b):
        pltpu.make_async_copy(idx_hbm.at[0, pl.ds(sbase, C)],
                              idx_v.at[b], idx_sem.at[b]).wait()

    def start_gather(b):
        pltpu.async_copy(table_hbm.at[idx_v.at[b]], rows_v.at[b],
                         gat_sem.at[b])

    def wait_gather(b):
        pltpu.make_async_copy(table_hbm.at[idx_v.at[b]], rows_v.at[b],
                              gat_sem.at[b]).wait()

    lgbase = wid * (C // 128)

    def start_out(j, b):
        for sg in range(DIM // 8):
            for lg in range(C // 128):
                pltpu.async_copy(
                    trows_v.at[b, pl.ds(sg * 8, 8), pl.ds(lg * 128, 128)],
                    out_hbm.at[j, sg, lgbase + lg, :, :],
                    out_sem.at[b])

    def wait_out(b):
        for _ in range((DIM // 8) * (C // 128)):
            pltpu.make_async_copy(
                trows_v.at[b, pl.ds(0, 8), pl.ds(0, 128)],
                out_hbm.at[0, 0, 0, :, :],
                out_sem.at[b]).wait()

    def transpose(b):
        rows = rows_v.at[b]
        trows = trows_v.at[b]
        d_lo = lax.iota(jnp.int32, L)
        d_hi = d_lo + L

        @plsc.parallel_loop(0, C // 4, 1, unroll=2)
        def _(i0):
            for u in range(4):
                i = i0 * 4 + u
                col = jnp.full((L,), 0, jnp.int32) + i
                v0 = rows[i, pl.ds(0, L)]
                v1 = rows[i, pl.ds(L, L)]
                plsc.store_scatter(trows, [d_lo, col], v0)
                plsc.store_scatter(trows, [d_hi, col], v1)

    for b in range(NBUF):
        start_idx(b, b)

    def outer(jj, carry):
        for b in range(NBUF):
            g = jj * NBUF + b

            @pl.when(jj > 0)
            def _():
                wait_out(b)

            wait_idx(b)
            start_gather(b)

            bp = (b - 1) % NBUF

            def retire(g_prev, bp=bp):
                wait_gather(bp)

                @pl.when(g_prev + NBUF < NCHUNK)
                def _():
                    start_idx(g_prev + NBUF, bp)

                transpose(bp)
                start_out(g_prev, bp)

            if b == 0:
                @pl.when(jj > 0)
                def _():
                    retire(jj * NBUF - 1)
            else:
                retire(g - 1)
        return carry

    lax.fori_loop(0, NCHUNK // NBUF, outer, 0)

    b_last = (NCHUNK - 1) % NBUF
    wait_gather(b_last)
    transpose(b_last)
    start_out(NCHUNK - 1, b_last)
    for b in range(NBUF):
        wait_out(b)


_sc_gather = functools.partial(
    pl.kernel,
    out_type=jax.ShapeDtypeStruct((COLS, DIM // 8, ROWS // 128, 8, 128),
                                  jnp.float32),
    mesh=plsc.VectorSubcoreMesh(core_axis_name="c", subcore_axis_name="s"),
    scratch_types=[
        pltpu.VMEM((NBUF, C), jnp.int32),
        pltpu.VMEM((NBUF, C, DIM), jnp.float32),
        pltpu.VMEM((NBUF, DIM, C + 1), jnp.float32),
        pltpu.SemaphoreType.DMA((NBUF,)),
        pltpu.SemaphoreType.DMA((NBUF,)),
        pltpu.SemaphoreType.DMA((NBUF,)),
    ],
    compiler_params=pltpu.CompilerParams(use_tc_tiling_on_sc=False,
                                         needs_layout_passes=False),
)(_sc_gather_body)


def kernel(input_tensor, emb_weight):
    idx_t = _hash_tc(input_tensor.T)
    table4 = jnp.pad(emb_weight, ((0, 0), (0, 96))).reshape(4 * NUM_EMB, DIM)
    out5 = _sc_gather(idx_t, table4)
    out_t = out5.transpose((0, 1, 3, 2, 4)).reshape(COLS, DIM, ROWS)
    return jnp.transpose(out_t, (2, 0, 1))
